# baseline (speedup 1.0000x reference)
import jax
import jax.numpy as jnp
from jax.experimental import pallas as pl
from jax.experimental.pallas import tpu as pltpu

T, D, E, K, F = 2048, 768, 64, 2, 512
EB = 2
NE = E // EB
CH = 128
NSP = 4736
TILE = 256
RC = 128


def _router_kernel(x_ref, wr_ref, w1_ref, w2_ref, p0_ref, p1_ref,
                   cnt_ref, poffs_ref):
    logits = jnp.dot(x_ref[...], wr_ref[...], preferred_element_type=jnp.float32)
    m = jnp.max(logits, axis=-1, keepdims=True)
    p = jnp.exp(logits - m)
    probs = p / jnp.sum(p, axis=-1, keepdims=True)
    iota = jax.lax.broadcasted_iota(jnp.int32, (T, E), 1)
    v1 = jnp.max(probs, axis=-1, keepdims=True)
    i1 = jnp.min(jnp.where(probs == v1, iota, E), axis=-1, keepdims=True)
    pm = jnp.where(iota == i1, -jnp.inf, probs)
    v2 = jnp.max(pm, axis=-1, keepdims=True)
    i2 = jnp.min(jnp.where(pm == v2, iota, E), axis=-1, keepdims=True)
    s = v1 + v2
    w1_ref[...] = v1 / s
    w2_ref[...] = v2 / s

    c1 = (iota == i1).astype(jnp.float32)
    c2 = (iota == i2).astype(jnp.float32)
    cnt = (jnp.sum(c1, axis=0, keepdims=True)
           + jnp.sum(c2, axis=0, keepdims=True))
    cnt_i = cnt.astype(jnp.int32)
    pcnt_i = (cnt_i + 7) // 8 * 8
    eiota_r = jax.lax.broadcasted_iota(jnp.int32, (E, E), 0)
    eiota_c = jax.lax.broadcasted_iota(jnp.int32, (E, E), 1)
    strict_ge = (eiota_r < eiota_c).astype(jnp.float32)
    hi = (pcnt_i // 256).astype(jnp.float32)
    lo = (pcnt_i % 256).astype(jnp.float32)
    poffs = (256.0 * jnp.dot(hi, strict_ge, preferred_element_type=jnp.float32)
             + jnp.dot(lo, strict_ge, preferred_element_type=jnp.float32))

    riota_r = jax.lax.broadcasted_iota(jnp.int32, (RC, RC), 0)
    riota_c = jax.lax.broadcasted_iota(jnp.int32, (RC, RC), 1)
    tri = (riota_r >= riota_c).astype(jnp.float32)

    def ranks_of(c, running):
        parts = []
        for b in range(T // RC):
            blk = c[b * RC:(b + 1) * RC, :]
            pref = jnp.dot(tri, blk, preferred_element_type=jnp.float32)
            parts.append(pref + running)
            running = running + pref[RC - 1:RC, :]
        prefix = jnp.concatenate(parts, axis=0)
        rank = jnp.sum(c * prefix, axis=-1, keepdims=True) - 1.0
        return rank, running

    zero_row = jnp.zeros((1, E), jnp.float32)
    rank1, running = ranks_of(c1, zero_row)
    rank2, _ = ranks_of(c2, running)

    start1 = jnp.sum(c1 * poffs, axis=-1, keepdims=True)
    start2 = jnp.sum(c2 * poffs, axis=-1, keepdims=True)
    p0_ref[...] = (start1 + rank1).astype(jnp.int32)
    p1_ref[...] = (start2 + rank2).astype(jnp.int32)
    cnt_ref[...] = cnt.astype(jnp.int32)
    poffs_ref[...] = poffs.astype(jnp.int32)


def _moe_kernel(poffs_ref, cnt_ref, p0_ref, p1_ref, x_ref, w1_ref, w2_ref,
                s0_ref, s1_ref, wn0_ref, wn1_ref, out_ref, ybuf_ref):
    g = pl.program_id(0)

    @pl.when(g == 0)
    def _init():
        ybuf_ref[...] = jnp.zeros((NSP, D), jnp.float32)

    @pl.when(g < NE)
    def _experts():
        iotac = jax.lax.broadcasted_iota(jnp.int32, (T, CH), 1)
        p0 = p0_ref[...]
        p1 = p1_ref[...]

        for j in range(EB):
            e = g * EB + j
            start = poffs_ref[e]
            n = cnt_ref[e]

            def body(c, _, start=start, n=n, j=j):
                base = pl.multiple_of(start + c * CH, 8)
                onehot_t = ((iotac == p0 - base) | (iotac == p1 - base)
                            ).astype(jnp.float32)
                xg = jax.lax.dot_general(
                    onehot_t, x_ref[...], (((0,), (0,)), ((), ())),
                    preferred_element_type=jnp.float32)
                h = jnp.maximum(
                    jnp.dot(xg, w1_ref[j], preferred_element_type=jnp.float32),
                    0.0)
                y = jnp.dot(h, w2_ref[j], preferred_element_type=jnp.float32)
                ybuf_ref[pl.ds(base, CH), :] = y
                return 0

            nchunks = (n + CH - 1) // CH
            jax.lax.fori_loop(0, nchunks, body, 0)

    @pl.when(g >= NE)
    def _combine():
        iota = jax.lax.broadcasted_iota(jnp.int32, (TILE, NSP), 1)
        gm = (jnp.where(iota == s0_ref[...], wn0_ref[...], 0.0)
              + jnp.where(iota == s1_ref[...], wn1_ref[...], 0.0))
        out_ref[...] = jnp.dot(gm, ybuf_ref[...],
                               preferred_element_type=jnp.float32)


def kernel(x, w_router, w1, w2):
    B, L, Dv = x.shape
    xf = x.reshape(B * L, Dv)

    w1n, w2n, p0, p1, cnt, poffs = pl.pallas_call(
        _router_kernel,
        out_shape=[
            jax.ShapeDtypeStruct((T, 1), jnp.float32),
            jax.ShapeDtypeStruct((T, 1), jnp.float32),
            jax.ShapeDtypeStruct((T, 1), jnp.int32),
            jax.ShapeDtypeStruct((T, 1), jnp.int32),
            jax.ShapeDtypeStruct((1, E), jnp.int32),
            jax.ShapeDtypeStruct((1, E), jnp.int32),
        ],
    )(xf, w_router)

    we = lambda g, po, cn: (jnp.minimum(g, NE - 1), 0, 0)
    ct = lambda g, po, cn: (jnp.maximum(g - NE, 0), 0)
    full = lambda g, po, cn: (0, 0)

    out = pl.pallas_call(
        _moe_kernel,
        grid_spec=pltpu.PrefetchScalarGridSpec(
            num_scalar_prefetch=2,
            grid=(NE + T // TILE,),
            in_specs=[
                pl.BlockSpec((T, 1), full),
                pl.BlockSpec((T, 1), full),
                pl.BlockSpec((T, D), full),
                pl.BlockSpec((EB, D, F), we),
                pl.BlockSpec((EB, F, D), we),
                pl.BlockSpec((TILE, 1), ct),
                pl.BlockSpec((TILE, 1), ct),
                pl.BlockSpec((TILE, 1), ct),
                pl.BlockSpec((TILE, 1), ct),
            ],
            out_specs=pl.BlockSpec((TILE, D), ct),
            scratch_shapes=[pltpu.VMEM((NSP, D), jnp.float32)],
        ),
        out_shape=jax.ShapeDtypeStruct((T, D), jnp.float32),
    )(poffs[0], cnt[0], p0, p1, xf, w1, w2, p0, p1, w1n, w2n)

    return out.reshape(B, L, Dv)


# speedup vs baseline: 1.3277x; 1.3277x over previous
import jax
import jax.numpy as jnp
from jax.experimental import pallas as pl
from jax.experimental.pallas import tpu as pltpu

T, D, E, K, F = 2048, 768, 64, 2, 512
EB = 2
NE = E // EB
CH = 128
NSP = 4736
TILE = 512
RC = 128


def _router_kernel(x_ref, wr_ref, w1_ref, w2_ref, p0_ref, p1_ref,
                   cnt_ref, poffs_ref):
    logits = jnp.dot(x_ref[...], wr_ref[...], preferred_element_type=jnp.float32)
    m = jnp.max(logits, axis=-1, keepdims=True)
    p = jnp.exp(logits - m)
    probs = p / jnp.sum(p, axis=-1, keepdims=True)
    iota = jax.lax.broadcasted_iota(jnp.int32, (T, E), 1)
    v1 = jnp.max(probs, axis=-1, keepdims=True)
    i1 = jnp.min(jnp.where(probs == v1, iota, E), axis=-1, keepdims=True)
    pm = jnp.where(iota == i1, -jnp.inf, probs)
    v2 = jnp.max(pm, axis=-1, keepdims=True)
    i2 = jnp.min(jnp.where(pm == v2, iota, E), axis=-1, keepdims=True)
    s = v1 + v2
    w1_ref[...] = v1 / s
    w2_ref[...] = v2 / s

    c1 = (iota == i1).astype(jnp.float32)
    c2 = (iota == i2).astype(jnp.float32)
    cnt = (jnp.sum(c1, axis=0, keepdims=True)
           + jnp.sum(c2, axis=0, keepdims=True))
    cnt_i = cnt.astype(jnp.int32)
    pcnt_i = (cnt_i + 7) // 8 * 8
    eiota_r = jax.lax.broadcasted_iota(jnp.int32, (E, E), 0)
    eiota_c = jax.lax.broadcasted_iota(jnp.int32, (E, E), 1)
    strict_ge = (eiota_r < eiota_c).astype(jnp.float32)
    hi = (pcnt_i // 256).astype(jnp.float32)
    lo = (pcnt_i % 256).astype(jnp.float32)
    poffs = (256.0 * jnp.dot(hi, strict_ge, preferred_element_type=jnp.float32)
             + jnp.dot(lo, strict_ge, preferred_element_type=jnp.float32))

    riota_r = jax.lax.broadcasted_iota(jnp.int32, (RC, RC), 0)
    riota_c = jax.lax.broadcasted_iota(jnp.int32, (RC, RC), 1)
    tri = (riota_r >= riota_c).astype(jnp.float32)

    def ranks_of(c, running):
        parts = []
        for b in range(T // RC):
            blk = c[b * RC:(b + 1) * RC, :]
            pref = jnp.dot(tri, blk, preferred_element_type=jnp.float32)
            parts.append(pref + running)
            running = running + pref[RC - 1:RC, :]
        prefix = jnp.concatenate(parts, axis=0)
        rank = jnp.sum(c * prefix, axis=-1, keepdims=True) - 1.0
        return rank, running

    zero_row = jnp.zeros((1, E), jnp.float32)
    rank1, running = ranks_of(c1, zero_row)
    rank2, _ = ranks_of(c2, running)

    start1 = jnp.sum(c1 * poffs, axis=-1, keepdims=True)
    start2 = jnp.sum(c2 * poffs, axis=-1, keepdims=True)
    p0_ref[...] = (start1 + rank1).astype(jnp.int32)
    p1_ref[...] = (start2 + rank2).astype(jnp.int32)
    cnt_ref[...] = cnt.astype(jnp.int32)
    poffs_ref[...] = poffs.astype(jnp.int32)


def _moe_kernel(poffs_ref, cnt_ref, p0_ref, p1_ref, x_ref, w1_ref, w2_ref,
                s0_ref, s1_ref, wn0_ref, wn1_ref, out_ref, ybuf_ref):
    g = pl.program_id(0)

    @pl.when(g == 0)
    def _init():
        ybuf_ref[...] = jnp.zeros((NSP, D), jnp.float32)

    @pl.when(g < NE)
    def _experts():
        iota0 = jax.lax.broadcasted_iota(jnp.int32, (CH, T), 0)
        p0 = p0_ref[...]
        p1 = p1_ref[...]

        for j in range(EB):
            e = g * EB + j
            start = poffs_ref[e]
            n = cnt_ref[e]

            def body(c, _, start=start, n=n, j=j):
                base = pl.multiple_of(start + c * CH, 8)
                onehot = ((iota0 == p0 - base) | (iota0 == p1 - base)
                          ).astype(jnp.float32)
                xg = jnp.dot(onehot, x_ref[...],
                             preferred_element_type=jnp.float32)
                h = jnp.maximum(
                    jnp.dot(xg, w1_ref[j], preferred_element_type=jnp.float32),
                    0.0)
                y = jnp.dot(h, w2_ref[j], preferred_element_type=jnp.float32)
                ybuf_ref[pl.ds(base, CH), :] = y
                return 0

            nchunks = (n + CH - 1) // CH
            jax.lax.fori_loop(0, nchunks, body, 0)

    @pl.when(g >= NE)
    def _combine():
        iota = jax.lax.broadcasted_iota(jnp.int32, (TILE, NSP), 1)
        gm = (jnp.where(iota == s0_ref[...], wn0_ref[...], 0.0)
              + jnp.where(iota == s1_ref[...], wn1_ref[...], 0.0))
        out_ref[...] = jnp.dot(gm, ybuf_ref[...],
                               preferred_element_type=jnp.float32)


def kernel(x, w_router, w1, w2):
    B, L, Dv = x.shape
    xf = x.reshape(B * L, Dv)

    w1n, w2n, p0, p1, cnt, poffs = pl.pallas_call(
        _router_kernel,
        out_shape=[
            jax.ShapeDtypeStruct((T, 1), jnp.float32),
            jax.ShapeDtypeStruct((T, 1), jnp.float32),
            jax.ShapeDtypeStruct((T, 1), jnp.int32),
            jax.ShapeDtypeStruct((T, 1), jnp.int32),
            jax.ShapeDtypeStruct((1, E), jnp.int32),
            jax.ShapeDtypeStruct((1, E), jnp.int32),
        ],
    )(xf, w_router)

    p0r = p0.reshape(1, T)
    p1r = p1.reshape(1, T)

    we = lambda g, po, cn: (jnp.minimum(g, NE - 1), 0, 0)
    ct = lambda g, po, cn: (jnp.maximum(g - NE, 0), 0)
    full = lambda g, po, cn: (0, 0)

    out = pl.pallas_call(
        _moe_kernel,
        grid_spec=pltpu.PrefetchScalarGridSpec(
            num_scalar_prefetch=2,
            grid=(NE + T // TILE,),
            in_specs=[
                pl.BlockSpec((1, T), full),
                pl.BlockSpec((1, T), full),
                pl.BlockSpec((T, D), full),
                pl.BlockSpec((EB, D, F), we),
                pl.BlockSpec((EB, F, D), we),
                pl.BlockSpec((TILE, 1), ct),
                pl.BlockSpec((TILE, 1), ct),
                pl.BlockSpec((TILE, 1), ct),
                pl.BlockSpec((TILE, 1), ct),
            ],
            out_specs=pl.BlockSpec((TILE, D), ct),
            scratch_shapes=[pltpu.VMEM((NSP, D), jnp.float32)],
        ),
        out_shape=jax.ShapeDtypeStruct((T, D), jnp.float32),
    )(poffs[0], cnt[0], p0r, p1r, xf, w1, w2, p0, p1, w1n, w2n)

    return out.reshape(B, L, Dv)
